# baseline (speedup 1.0000x reference)
import functools

import jax
import jax.numpy as jnp
from jax import lax
from jax.experimental import pallas as pl
from jax.experimental.pallas import tpu as pltpu
from jax.experimental.pallas import tpu_sc as plsc

B = 256
C = 3
H = W = 64
HW = H * W
D = 64
V = C * 256
NC, NS = 2, 16
NW = NC * NS
BPW = B // NW
CHUNK = 256
STR = CHUNK + 1
NCHUNK = HW // CHUNK
PG = 16
NQ = D // 16

_mesh = plsc.VectorSubcoreMesh(core_axis_name="c", subcore_axis_name="s")


@functools.partial(
    pl.kernel,
    mesh=_mesh,
    out_type=jax.ShapeDtypeStruct((B, D, HW), jnp.float32),
    scratch_types=[
        pltpu.VMEM((V * D,), jnp.float32),
        pltpu.VMEM((C, CHUNK), jnp.int32),
        pltpu.VMEM((D, STR), jnp.float32),
    ],
    compiler_params=pltpu.CompilerParams(needs_layout_passes=False),
)
def _bow_sc(x_hbm, table_hbm, out_hbm, table_v, x_v, o_v):
    wid = lax.axis_index("s") * NC + lax.axis_index("c")
    pltpu.sync_copy(table_hbm, table_v)

    iota = lax.iota(jnp.int32, 16)
    dqc = [[iota + (c * 256 * D + 16 * q) for q in range(NQ)]
           for c in range(C)]
    dq = [iota + 16 * q for q in range(NQ)]

    def batch_body(i, carry):
        b = wid * BPW + i

        def chunk_body(k, carry):
            pltpu.sync_copy(x_hbm.at[b, :, pl.ds(k * CHUNK, CHUNK)], x_v)

            def grp_body(g, carry):
                off = g * PG
                rb0 = x_v[0, pl.ds(off, PG)] * D
                rb1 = x_v[1, pl.ds(off, PG)] * D
                rb2 = x_v[2, pl.ds(off, PG)] * D
                for u in range(PG):
                    p = off + u
                    r0, r1, r2 = rb0[u], rb1[u], rb2[u]
                    p_vec = jnp.broadcast_to(p, (16,)).astype(jnp.int32)
                    for q in range(NQ):
                        acc = (plsc.load_gather(table_v, [r0 + dqc[0][q]])
                               + plsc.load_gather(table_v, [r1 + dqc[1][q]])
                               + plsc.load_gather(table_v, [r2 + dqc[2][q]]))
                        plsc.store_scatter(o_v, [dq[q], p_vec], acc)
                return carry

            lax.fori_loop(0, CHUNK // PG, grp_body, 0)
            pltpu.sync_copy(o_v.at[:, pl.ds(0, CHUNK)],
                            out_hbm.at[b, :, pl.ds(k * CHUNK, CHUNK)])
            return carry

        lax.fori_loop(0, NCHUNK, chunk_body, 0)
        return carry

    lax.fori_loop(0, BPW, batch_body, 0)


def kernel(x, table):
    x3 = x.reshape(B, C, HW).astype(jnp.int32)
    out = _bow_sc(x3, table.reshape(-1))
    return out.reshape(B, D, H, W)


# speedup vs baseline: 2.1703x; 2.1703x over previous
import functools

import jax
import jax.numpy as jnp
from jax import lax
from jax.experimental import pallas as pl
from jax.experimental.pallas import tpu as pltpu
from jax.experimental.pallas import tpu_sc as plsc

B = 256
C = 3
H = W = 64
HW = H * W
D = 64
V = C * 256
WROW = D // 2
WSTR = WROW + 1
NC, NS = 2, 16
NW = NC * NS
BPW = B // NW
CHUNK = 256
NCHUNK = HW // CHUNK
NPB = CHUNK // 16

_mesh = plsc.VectorSubcoreMesh(core_axis_name="c", subcore_axis_name="s")


@functools.partial(
    pl.kernel,
    mesh=_mesh,
    out_type=jax.ShapeDtypeStruct((B, D, HW), jnp.float32),
    scratch_types=[
        pltpu.VMEM((V * WSTR,), jnp.int32),
        pltpu.VMEM((C, HW), jnp.int32),
        pltpu.VMEM((D, CHUNK), jnp.float32),
    ],
    compiler_params=pltpu.CompilerParams(needs_layout_passes=False),
)
def _bow_sc(x_hbm, tw_hbm, out_hbm, table_v, x_v, o_v):
    wid = lax.axis_index("s") * NC + lax.axis_index("c")
    pltpu.sync_copy(tw_hbm, table_v)

    one = jnp.ones((16,), jnp.int32)
    himask = jnp.full((16,), -65536, jnp.int32)

    def batch_body(i, carry):
        b = wid * BPW + i
        pltpu.sync_copy(x_hbm.at[b], x_v)

        def chunk_body(k, carry):
            def pb_body(pb, carry):
                off = k * CHUNK + pb * 16
                i0 = x_v[0, pl.ds(off, 16)] * WSTR
                i1 = x_v[1, pl.ds(off, 16)] * WSTR + 256 * WSTR
                i2 = x_v[2, pl.ds(off, 16)] * WSTR + 512 * WSTR
                for q in range(WROW):
                    w0 = plsc.load_gather(table_v, [i0])
                    w1 = plsc.load_gather(table_v, [i1])
                    w2 = plsc.load_gather(table_v, [i2])
                    acc = (plsc.bitcast(w0, jnp.bfloat16)
                           + plsc.bitcast(w1, jnp.bfloat16)
                           + plsc.bitcast(w2, jnp.bfloat16))
                    accw = plsc.bitcast(acc, jnp.int32)
                    lo = plsc.bitcast(accw << 16, jnp.float32)
                    hi = plsc.bitcast(accw & himask, jnp.float32)
                    o_v[2 * q, pl.ds(pb * 16, 16)] = lo
                    o_v[2 * q + 1, pl.ds(pb * 16, 16)] = hi
                    if q + 1 < WROW:
                        i0 = i0 + one
                        i1 = i1 + one
                        i2 = i2 + one
                return carry

            lax.fori_loop(0, NPB, pb_body, 0)
            pltpu.sync_copy(o_v, out_hbm.at[b, :, pl.ds(k * CHUNK, CHUNK)])
            return carry

        lax.fori_loop(0, NCHUNK, chunk_body, 0)
        return carry

    lax.fori_loop(0, BPW, batch_body, 0)


def kernel(x, table):
    x3 = x.reshape(B, C, HW).astype(jnp.int32)
    tb = table.astype(jnp.bfloat16).reshape(V, WROW, 2)
    tw = jax.lax.bitcast_convert_type(tb, jnp.int32)
    tw = jnp.pad(tw, ((0, 0), (0, WSTR - WROW))).reshape(-1)
    out = _bow_sc(x3, tw)
    return out.reshape(B, D, H, W)


# speedup vs baseline: 2.1763x; 1.0028x over previous
import functools

import jax
import jax.numpy as jnp
from jax import lax
from jax.experimental import pallas as pl
from jax.experimental.pallas import tpu as pltpu
from jax.experimental.pallas import tpu_sc as plsc

B = 256
C = 3
H = W = 64
HW = H * W
D = 64
V = C * 256
WROW = D // 2
NC, NS = 2, 16
NW = NC * NS
BPW = B // NW
CHUNK = 256
NCHUNK = HW // CHUNK
NPB = CHUNK // 16

_mesh = plsc.VectorSubcoreMesh(core_axis_name="c", subcore_axis_name="s")


@functools.partial(
    pl.kernel,
    mesh=_mesh,
    out_type=jax.ShapeDtypeStruct((B, D, HW), jnp.float32),
    scratch_types=[
        pltpu.VMEM((WROW * V,), jnp.int32),
        pltpu.VMEM((C, HW), jnp.int32),
        pltpu.VMEM((D, CHUNK), jnp.float32),
    ],
    compiler_params=pltpu.CompilerParams(needs_layout_passes=False),
)
def _bow_sc(x_hbm, tw_hbm, out_hbm, table_v, x_v, o_v):
    wid = lax.axis_index("s") * NC + lax.axis_index("c")
    pltpu.sync_copy(tw_hbm, table_v)

    himask = jnp.full((16,), -65536, jnp.int32)

    def batch_body(i, carry):
        b = wid * BPW + i
        pltpu.sync_copy(x_hbm.at[b], x_v)

        def chunk_body(k, carry):
            def pb_body(pb, carry):
                off = k * CHUNK + pb * 16
                i0 = x_v[0, pl.ds(off, 16)]
                i1 = x_v[1, pl.ds(off, 16)] + 256
                i2 = x_v[2, pl.ds(off, 16)] + 512
                for q in range(WROW):
                    tq = table_v.at[pl.ds(q * V, V)]
                    w0 = plsc.load_gather(tq, [i0])
                    w1 = plsc.load_gather(tq, [i1])
                    w2 = plsc.load_gather(tq, [i2])
                    acc = (plsc.bitcast(w0, jnp.bfloat16)
                           + plsc.bitcast(w1, jnp.bfloat16)
                           + plsc.bitcast(w2, jnp.bfloat16))
                    accw = plsc.bitcast(acc, jnp.int32)
                    lo = plsc.bitcast(accw << 16, jnp.float32)
                    hi = plsc.bitcast(accw & himask, jnp.float32)
                    o_v[2 * q, pl.ds(pb * 16, 16)] = lo
                    o_v[2 * q + 1, pl.ds(pb * 16, 16)] = hi
                return carry

            lax.fori_loop(0, NPB, pb_body, 0)
            pltpu.sync_copy(o_v, out_hbm.at[b, :, pl.ds(k * CHUNK, CHUNK)])
            return carry

        lax.fori_loop(0, NCHUNK, chunk_body, 0)
        return carry

    lax.fori_loop(0, BPW, batch_body, 0)


def kernel(x, table):
    x3 = x.reshape(B, C, HW).astype(jnp.int32)
    tb = table.astype(jnp.bfloat16).reshape(V, WROW, 2)
    tw = jax.lax.bitcast_convert_type(tb, jnp.int32)
    tw = tw.T.reshape(-1)
    out = _bow_sc(x3, tw)
    return out.reshape(B, D, H, W)


# speedup vs baseline: 2.3708x; 1.0894x over previous
import functools

import jax
import jax.numpy as jnp
from jax import lax
from jax.experimental import pallas as pl
from jax.experimental.pallas import tpu as pltpu
from jax.experimental.pallas import tpu_sc as plsc

B = 256
C = 3
H = W = 64
HW = H * W
D = 64
V = C * 256
WROW = D // 2
NC, NS = 2, 16
NW = NC * NS
BPW = B // NW
CHUNK = 256
NCHUNK = HW // CHUNK
NPB = CHUNK // 16

_mesh = plsc.VectorSubcoreMesh(core_axis_name="c", subcore_axis_name="s")


@functools.partial(
    pl.kernel,
    mesh=_mesh,
    out_type=jax.ShapeDtypeStruct((B, D, HW), jnp.float32),
    scratch_types=[
        pltpu.VMEM((WROW * V,), jnp.int32),
        pltpu.VMEM((C, HW), jnp.int32),
        pltpu.VMEM((D, CHUNK), jnp.float32),
        pltpu.VMEM((D, CHUNK), jnp.float32),
        pltpu.SemaphoreType.DMA,
        pltpu.SemaphoreType.DMA,
    ],
    compiler_params=pltpu.CompilerParams(needs_layout_passes=False),
)
def _bow_sc(x_hbm, tw_hbm, out_hbm, table_v, x_v, o0, o1, os0, os1):
    o_b = [o0, o1]
    osem = [os0, os1]
    wid = lax.axis_index("s") * NC + lax.axis_index("c")
    pltpu.sync_copy(tw_hbm, table_v)

    himask = jnp.full((16,), -65536, jnp.int32)
    NT = BPW * NCHUNK

    def out_desc(t, j):
        b = wid * BPW + t // NCHUNK
        k = t % NCHUNK
        return pltpu.make_async_copy(
            o_b[j], out_hbm.at[b, :, pl.ds(k * CHUNK, CHUNK)], osem[j])

    def task_body(t, carry):
        k = t % NCHUNK
        for j in range(2):
            @pl.when(t % 2 == j)
            def _():
                @pl.when(k == 0)
                def _():
                    b = wid * BPW + t // NCHUNK
                    pltpu.sync_copy(x_hbm.at[b], x_v)
                @pl.when(t >= 2)
                def _():
                    out_desc(t - 2, j).wait()
                o_v = o_b[j]

                def pb_body(pb, carry):
                    off = k * CHUNK + pb * 16
                    i0 = x_v[0, pl.ds(off, 16)]
                    i1 = x_v[1, pl.ds(off, 16)] + 256
                    i2 = x_v[2, pl.ds(off, 16)] + 512
                    for q in range(WROW):
                        tq = table_v.at[pl.ds(q * V, V)]
                        w0 = plsc.load_gather(tq, [i0])
                        w1 = plsc.load_gather(tq, [i1])
                        w2 = plsc.load_gather(tq, [i2])
                        acc = (plsc.bitcast(w0, jnp.bfloat16)
                               + plsc.bitcast(w1, jnp.bfloat16)
                               + plsc.bitcast(w2, jnp.bfloat16))
                        accw = plsc.bitcast(acc, jnp.int32)
                        lo = plsc.bitcast(accw << 16, jnp.float32)
                        hi = plsc.bitcast(accw & himask, jnp.float32)
                        o_v[2 * q, pl.ds(pb * 16, 16)] = lo
                        o_v[2 * q + 1, pl.ds(pb * 16, 16)] = hi
                    return carry

                lax.fori_loop(0, NPB, pb_body, 0)
                out_desc(t, j).start()
        return carry

    lax.fori_loop(0, NT, task_body, 0)
    out_desc(NT - 2, (NT - 2) % 2).wait()
    out_desc(NT - 1, (NT - 1) % 2).wait()


def kernel(x, table):
    x3 = x.reshape(B, C, HW).astype(jnp.int32)
    tb = table.astype(jnp.bfloat16).reshape(V, WROW, 2)
    tw = jax.lax.bitcast_convert_type(tb, jnp.int32)
    tw = tw.T.reshape(-1)
    out = _bow_sc(x3, tw)
    return out.reshape(B, D, H, W)


# speedup vs baseline: 3.5009x; 1.4767x over previous
import functools

import jax
import jax.numpy as jnp
from jax import lax
from jax.experimental import pallas as pl
from jax.experimental.pallas import tpu as pltpu
from jax.experimental.pallas import tpu_sc as plsc

B = 256
C = 3
H = W = 64
HW = H * W
D = 64
V = C * 256
WROW = D // 2
NC, NS = 2, 16
NW = NC * NS
BPW = B // NW
CHUNK = 256
NCHUNK = HW // CHUNK
NPB = CHUNK // 16

_mesh = plsc.VectorSubcoreMesh(core_axis_name="c", subcore_axis_name="s")


@functools.partial(
    pl.kernel,
    mesh=_mesh,
    out_type=jax.ShapeDtypeStruct((B, D, HW), jnp.float32),
    scratch_types=[
        pltpu.VMEM((WROW * V,), jnp.int32),
        pltpu.VMEM((C, HW), jnp.int32),
        pltpu.VMEM((D, CHUNK), jnp.float32),
        pltpu.VMEM((D, CHUNK), jnp.float32),
        pltpu.SemaphoreType.DMA,
        pltpu.SemaphoreType.DMA,
    ],
    compiler_params=pltpu.CompilerParams(needs_layout_passes=False),
)
def _bow_sc(x_hbm, tw_hbm, out_hbm, table_v, x_v, o0, o1, os0, os1):
    o_b = [o0, o1]
    osem = [os0, os1]
    wid = lax.axis_index("s") * NC + lax.axis_index("c")
    pltpu.sync_copy(tw_hbm, table_v)

    himask = jnp.full((16,), -65536, jnp.int32)
    NT = BPW * NCHUNK

    def out_desc(t, j):
        b = wid * BPW + t // NCHUNK
        k = t % NCHUNK
        return pltpu.make_async_copy(
            o_b[j], out_hbm.at[b, :, pl.ds(k * CHUNK, CHUNK)], osem[j])

    def task_body(t, carry):
        k = t % NCHUNK
        for j in range(2):
            @pl.when(t % 2 == j)
            def _():
                @pl.when(k == 0)
                def _():
                    b = wid * BPW + t // NCHUNK
                    pltpu.sync_copy(x_hbm.at[b], x_v)
                @pl.when(t >= 2)
                def _():
                    out_desc(t - 2, j).wait()
                o_v = o_b[j]

                @plsc.parallel_loop(0, NPB, 1, unroll=2)
                def pb_body(pb):
                    off = k * CHUNK + pb * 16
                    i0 = x_v[0, pl.ds(off, 16)]
                    i1 = x_v[1, pl.ds(off, 16)] + 256
                    i2 = x_v[2, pl.ds(off, 16)] + 512
                    for q in range(WROW):
                        tq = table_v.at[pl.ds(q * V, V)]
                        w0 = plsc.load_gather(tq, [i0])
                        w1 = plsc.load_gather(tq, [i1])
                        w2 = plsc.load_gather(tq, [i2])
                        acc = (plsc.bitcast(w0, jnp.bfloat16)
                               + plsc.bitcast(w1, jnp.bfloat16)
                               + plsc.bitcast(w2, jnp.bfloat16))
                        accw = plsc.bitcast(acc, jnp.int32)
                        lo = plsc.bitcast(accw << 16, jnp.float32)
                        hi = plsc.bitcast(accw & himask, jnp.float32)
                        o_v[2 * q, pl.ds(pb * 16, 16)] = lo
                        o_v[2 * q + 1, pl.ds(pb * 16, 16)] = hi

                out_desc(t, j).start()
        return carry

    lax.fori_loop(0, NT, task_body, 0)
    out_desc(NT - 2, (NT - 2) % 2).wait()
    out_desc(NT - 1, (NT - 1) % 2).wait()


def kernel(x, table):
    x3 = x.reshape(B, C, HW).astype(jnp.int32)
    tb = table.astype(jnp.bfloat16).reshape(V, WROW, 2)
    tw = jax.lax.bitcast_convert_type(tb, jnp.int32)
    tw = tw.T.reshape(-1)
    out = _bow_sc(x3, tw)
    return out.reshape(B, D, H, W)


# speedup vs baseline: 3.5055x; 1.0013x over previous
import functools

import jax
import jax.numpy as jnp
from jax import lax
from jax.experimental import pallas as pl
from jax.experimental.pallas import tpu as pltpu
from jax.experimental.pallas import tpu_sc as plsc

B = 256
C = 3
H = W = 64
HW = H * W
D = 64
V = C * 256
WROW = D // 2
NC, NS = 2, 16
NW = NC * NS
BPW = B // NW
CHUNK = 256
NCHUNK = HW // CHUNK
NPB = CHUNK // 16

_mesh = plsc.VectorSubcoreMesh(core_axis_name="c", subcore_axis_name="s")


@functools.partial(
    pl.kernel,
    mesh=_mesh,
    out_type=jax.ShapeDtypeStruct((B, D, HW), jnp.float32),
    scratch_types=[
        pltpu.VMEM((WROW * V,), jnp.int32),
        pltpu.VMEM((C, HW), jnp.int32),
        pltpu.VMEM((D, CHUNK), jnp.float32),
        pltpu.VMEM((D, CHUNK), jnp.float32),
        pltpu.SemaphoreType.DMA,
        pltpu.SemaphoreType.DMA,
    ],
    compiler_params=pltpu.CompilerParams(needs_layout_passes=False),
)
def _bow_sc(x_hbm, tw_hbm, out_hbm, table_v, x_v, o0, o1, os0, os1):
    o_b = [o0, o1]
    osem = [os0, os1]
    wid = lax.axis_index("s") * NC + lax.axis_index("c")
    pltpu.sync_copy(tw_hbm, table_v)

    himask = jnp.full((16,), -65536, jnp.int32)
    NT = BPW * NCHUNK

    def out_desc(t, j):
        b = wid * BPW + t // NCHUNK
        k = t % NCHUNK
        return pltpu.make_async_copy(
            o_b[j], out_hbm.at[b, :, pl.ds(k * CHUNK, CHUNK)], osem[j])

    def task_body(t, carry):
        k = t % NCHUNK
        for j in range(2):
            @pl.when(t % 2 == j)
            def _():
                @pl.when(k == 0)
                def _():
                    b = wid * BPW + t // NCHUNK
                    pltpu.sync_copy(x_hbm.at[b], x_v)
                @pl.when(t >= 2)
                def _():
                    out_desc(t - 2, j).wait()
                o_v = o_b[j]

                @plsc.parallel_loop(0, NPB, 1, unroll=4)
                def pb_body(pb):
                    off = k * CHUNK + pb * 16
                    i0 = x_v[0, pl.ds(off, 16)]
                    i1 = x_v[1, pl.ds(off, 16)] + 256
                    i2 = x_v[2, pl.ds(off, 16)] + 512
                    for q in range(WROW):
                        tq = table_v.at[pl.ds(q * V, V)]
                        w0 = plsc.load_gather(tq, [i0])
                        w1 = plsc.load_gather(tq, [i1])
                        w2 = plsc.load_gather(tq, [i2])
                        acc = (plsc.bitcast(w0, jnp.bfloat16)
                               + plsc.bitcast(w1, jnp.bfloat16)
                               + plsc.bitcast(w2, jnp.bfloat16))
                        accw = plsc.bitcast(acc, jnp.int32)
                        lo = plsc.bitcast(accw << 16, jnp.float32)
                        hi = plsc.bitcast(accw & himask, jnp.float32)
                        o_v[2 * q, pl.ds(pb * 16, 16)] = lo
                        o_v[2 * q + 1, pl.ds(pb * 16, 16)] = hi

                out_desc(t, j).start()
        return carry

    lax.fori_loop(0, NT, task_body, 0)
    out_desc(NT - 2, (NT - 2) % 2).wait()
    out_desc(NT - 1, (NT - 1) % 2).wait()


def kernel(x, table):
    x3 = x.reshape(B, C, HW).astype(jnp.int32)
    tb = table.astype(jnp.bfloat16).reshape(V, WROW, 2)
    tw = jax.lax.bitcast_convert_type(tb, jnp.int32)
    tw = tw.T.reshape(-1)
    out = _bow_sc(x3, tw)
    return out.reshape(B, D, H, W)
